# baseline (speedup 1.0000x reference)
import functools

import jax
import jax.numpy as jnp
from jax import lax
from jax.experimental import pallas as pl
from jax.experimental.pallas import tpu as pltpu
from jax.experimental.pallas import tpu_sc as plsc

BATCH = 16384
HIDDEN = 64
WIDE = 2 * HIDDEN

NUM_CORES = 2
NUM_SUBCORES = 16
NUM_WORKERS = NUM_CORES * NUM_SUBCORES
B_PER_W = BATCH // NUM_WORKERS
CHUNK = 128
N_CHUNKS = B_PER_W // CHUNK

_SC_MESH = plsc.VectorSubcoreMesh(core_axis_name="c", subcore_axis_name="s")


@functools.partial(
    pl.kernel,
    mesh=_SC_MESH,
    out_type=[
        jax.ShapeDtypeStruct((BATCH, WIDE), jnp.float32),
        jax.ShapeDtypeStruct((BATCH, WIDE), jnp.float32),
    ],
    scratch_types=[
        pltpu.VMEM((N_CHUNKS, CHUNK), jnp.int32),
        pltpu.VMEM((N_CHUNKS, CHUNK), jnp.int32),
        pltpu.VMEM((CHUNK, WIDE), jnp.float32),
        pltpu.VMEM((CHUNK, WIDE), jnp.float32),
        pltpu.VMEM((CHUNK, WIDE), jnp.float32),
        pltpu.VMEM((CHUNK, WIDE), jnp.float32),
        pltpu.SemaphoreType.DMA,
        pltpu.SemaphoreType.DMA,
    ],
)
def _sc_gather(u_idx_hbm, i_idx_hbm, ut_hbm, it_hbm, uo_hbm, io_hbm,
               uidx_v, iidx_v, urows0, urows1, irows0, irows1, sem_u, sem_i):
    wid = lax.axis_index("s") * NUM_CORES + lax.axis_index("c")
    base = wid * B_PER_W
    pltpu.sync_copy(u_idx_hbm.at[pl.ds(wid * N_CHUNKS, N_CHUNKS)], uidx_v)
    pltpu.sync_copy(i_idx_hbm.at[pl.ds(wid * N_CHUNKS, N_CHUNKS)], iidx_v)
    ubufs = (urows0, urows1)
    ibufs = (irows0, irows1)
    gathers = [None, None]
    for j in range(N_CHUNKS):
        b = j % 2
        gathers[b] = (
            pltpu.async_copy(ut_hbm.at[uidx_v.at[j]], ubufs[b], sem_u),
            pltpu.async_copy(it_hbm.at[iidx_v.at[j]], ibufs[b], sem_i),
        )
        if j > 0:
            pb = (j - 1) % 2
            off = base + (j - 1) * CHUNK
            pltpu.sync_copy(ubufs[pb], uo_hbm.at[pl.ds(off, CHUNK)])
            pltpu.sync_copy(ibufs[pb], io_hbm.at[pl.ds(off, CHUNK)])
        gu, gi = gathers[b]
        gu.wait()
        gi.wait()
    lb = (N_CHUNKS - 1) % 2
    off = base + (N_CHUNKS - 1) * CHUNK
    pltpu.sync_copy(ubufs[lb], uo_hbm.at[pl.ds(off, CHUNK)])
    pltpu.sync_copy(ibufs[lb], io_hbm.at[pl.ds(off, CHUNK)])


_MLP_BLOCK = 2048


def _mlp_body(uw, iw, pu, pi, w1u, w1v, b1, w2, b2, w3, b3, o):
    ue = jnp.where(pu[...] > 0, uw[...][:, HIDDEN:], uw[...][:, :HIDDEN])
    ie = jnp.where(pi[...] > 0, iw[...][:, HIDDEN:], iw[...][:, :HIDDEN])
    h = (jnp.dot(ue, w1u[...], preferred_element_type=jnp.float32)
         + jnp.dot(ie, w1v[...], preferred_element_type=jnp.float32)
         + b1[...])
    h = jnp.maximum(h, 0.0)
    h = jnp.dot(h, w2[...], preferred_element_type=jnp.float32) + b2[...]
    h = jnp.maximum(h, 0.0)
    z = jnp.dot(h, w3[...], preferred_element_type=jnp.float32) + b3[...]
    o[...] = jax.nn.sigmoid(z)


def _mlp(uw, iw, pu, pi, w1u, w1v, b1, w2, b2, w3, b3):
    nb = BATCH // _MLP_BLOCK
    const = lambda *_: (0, 0)
    return pl.pallas_call(
        _mlp_body,
        grid=(nb,),
        in_specs=[
            pl.BlockSpec((_MLP_BLOCK, WIDE), lambda i: (i, 0)),
            pl.BlockSpec((_MLP_BLOCK, WIDE), lambda i: (i, 0)),
            pl.BlockSpec((_MLP_BLOCK, 1), lambda i: (i, 0)),
            pl.BlockSpec((_MLP_BLOCK, 1), lambda i: (i, 0)),
            pl.BlockSpec((HIDDEN, HIDDEN), const),
            pl.BlockSpec((HIDDEN, HIDDEN), const),
            pl.BlockSpec((1, HIDDEN), const),
            pl.BlockSpec((HIDDEN, HIDDEN // 2), const),
            pl.BlockSpec((1, HIDDEN // 2), const),
            pl.BlockSpec((HIDDEN // 2, 1), const),
            pl.BlockSpec((1, 1), const),
        ],
        out_specs=pl.BlockSpec((_MLP_BLOCK, 1), lambda i: (i, 0)),
        out_shape=jax.ShapeDtypeStruct((BATCH, 1), jnp.float32),
    )(uw, iw, pu, pi, w1u, w1v, b1, w2, b2, w3, b3)


def kernel(user, item, user_table, item_table, W1, b1, W2, b2, W3, b3):
    user = user.astype(jnp.int32)
    item = item.astype(jnp.int32)
    u2d = (user // 2).reshape(BATCH // CHUNK, CHUNK)
    i2d = (item // 2).reshape(BATCH // CHUNK, CHUNK)
    ut_w = user_table.reshape(-1, WIDE)
    it_w = item_table.reshape(-1, WIDE)
    uw, iw = _sc_gather(u2d, i2d, ut_w, it_w)
    pu = (user % 2).astype(jnp.float32).reshape(BATCH, 1)
    pi = (item % 2).astype(jnp.float32).reshape(BATCH, 1)
    w1u = W1[:HIDDEN]
    w1v = W1[HIDDEN:]
    return _mlp(uw, iw, pu, pi, w1u, w1v,
                b1.reshape(1, HIDDEN), W2, b2.reshape(1, HIDDEN // 2),
                W3, b3.reshape(1, 1))


# speedup vs baseline: 1.0099x; 1.0099x over previous
import functools

import jax
import jax.numpy as jnp
from jax import lax
from jax.experimental import pallas as pl
from jax.experimental.pallas import tpu as pltpu
from jax.experimental.pallas import tpu_sc as plsc

BATCH = 16384
HIDDEN = 64

NUM_CORES = 2
NUM_SUBCORES = 16
NUM_WORKERS = NUM_CORES * NUM_SUBCORES
B_PER_W = BATCH // NUM_WORKERS
CHUNK = 128
N_CHUNKS = B_PER_W // CHUNK

_SC_MESH = plsc.VectorSubcoreMesh(core_axis_name="c", subcore_axis_name="s")


@functools.partial(
    pl.kernel,
    mesh=_SC_MESH,
    out_type=[
        jax.ShapeDtypeStruct((BATCH, HIDDEN), jnp.float32),
        jax.ShapeDtypeStruct((BATCH, HIDDEN), jnp.float32),
    ],
    scratch_types=[
        pltpu.VMEM((N_CHUNKS, CHUNK), jnp.int32),
        pltpu.VMEM((N_CHUNKS, CHUNK), jnp.int32),
        pltpu.VMEM((CHUNK, HIDDEN), jnp.float32),
        pltpu.VMEM((CHUNK, HIDDEN), jnp.float32),
        pltpu.VMEM((CHUNK, HIDDEN), jnp.float32),
        pltpu.VMEM((CHUNK, HIDDEN), jnp.float32),
        pltpu.SemaphoreType.DMA,
        pltpu.SemaphoreType.DMA,
        pltpu.SemaphoreType.DMA,
        pltpu.SemaphoreType.DMA,
    ],
    compiler_params=pltpu.CompilerParams(use_tc_tiling_on_sc=False),
)
def _sc_gather(u_idx_hbm, i_idx_hbm, ut_hbm, it_hbm, uo_hbm, io_hbm,
               uidx_v, iidx_v, urows0, urows1, irows0, irows1,
               sem_u0, sem_u1, sem_i0, sem_i1):
    wid = lax.axis_index("s") * NUM_CORES + lax.axis_index("c")
    base = wid * B_PER_W
    pltpu.sync_copy(u_idx_hbm.at[pl.ds(wid * N_CHUNKS, N_CHUNKS)], uidx_v)
    pltpu.sync_copy(i_idx_hbm.at[pl.ds(wid * N_CHUNKS, N_CHUNKS)], iidx_v)

    ubufs = (urows0, urows1)
    ibufs = (irows0, irows1)
    usems = (sem_u0, sem_u1)
    isems = (sem_i0, sem_i1)
    gathers = [None, None]
    for j in range(N_CHUNKS):
        b = j % 2
        gathers[b] = (
            pltpu.async_copy(ut_hbm.at[uidx_v.at[j]], ubufs[b], usems[b]),
            pltpu.async_copy(it_hbm.at[iidx_v.at[j]], ibufs[b], isems[b]),
        )
        if j > 0:
            p = (j - 1) % 2
            gu, gi = gathers[p]
            gu.wait()
            gi.wait()
            off = base + (j - 1) * CHUNK
            pltpu.sync_copy(ubufs[p], uo_hbm.at[pl.ds(off, CHUNK)])
            pltpu.sync_copy(ibufs[p], io_hbm.at[pl.ds(off, CHUNK)])
    lb = (N_CHUNKS - 1) % 2
    gu, gi = gathers[lb]
    gu.wait()
    gi.wait()
    off = base + (N_CHUNKS - 1) * CHUNK
    pltpu.sync_copy(ubufs[lb], uo_hbm.at[pl.ds(off, CHUNK)])
    pltpu.sync_copy(ibufs[lb], io_hbm.at[pl.ds(off, CHUNK)])


_MLP_BLOCK = 2048


def _mlp_body(ue, ie, w1u, w1v, b1, w2, b2, w3, b3, o):
    h = (jnp.dot(ue[...], w1u[...], preferred_element_type=jnp.float32)
         + jnp.dot(ie[...], w1v[...], preferred_element_type=jnp.float32)
         + b1[...])
    h = jnp.maximum(h, 0.0)
    h = jnp.dot(h, w2[...], preferred_element_type=jnp.float32) + b2[...]
    h = jnp.maximum(h, 0.0)
    z = jnp.dot(h, w3[...], preferred_element_type=jnp.float32) + b3[...]
    o[...] = jax.nn.sigmoid(z)


def _mlp(ue, ie, w1u, w1v, b1, w2, b2, w3, b3):
    nb = BATCH // _MLP_BLOCK
    const = lambda *_: (0, 0)
    return pl.pallas_call(
        _mlp_body,
        grid=(nb,),
        in_specs=[
            pl.BlockSpec((_MLP_BLOCK, HIDDEN), lambda i: (i, 0)),
            pl.BlockSpec((_MLP_BLOCK, HIDDEN), lambda i: (i, 0)),
            pl.BlockSpec((HIDDEN, HIDDEN), const),
            pl.BlockSpec((HIDDEN, HIDDEN), const),
            pl.BlockSpec((1, HIDDEN), const),
            pl.BlockSpec((HIDDEN, HIDDEN // 2), const),
            pl.BlockSpec((1, HIDDEN // 2), const),
            pl.BlockSpec((HIDDEN // 2, 1), const),
            pl.BlockSpec((1, 1), const),
        ],
        out_specs=pl.BlockSpec((_MLP_BLOCK, 1), lambda i: (i, 0)),
        out_shape=jax.ShapeDtypeStruct((BATCH, 1), jnp.float32),
    )(ue, ie, w1u, w1v, b1, w2, b2, w3, b3)


def kernel(user, item, user_table, item_table, W1, b1, W2, b2, W3, b3):
    u2d = user.astype(jnp.int32).reshape(BATCH // CHUNK, CHUNK)
    i2d = item.astype(jnp.int32).reshape(BATCH // CHUNK, CHUNK)
    ue, ie = _sc_gather(u2d, i2d, user_table, item_table)
    w1u = W1[:HIDDEN]
    w1v = W1[HIDDEN:]
    return _mlp(ue, ie, w1u, w1v,
                b1.reshape(1, HIDDEN), W2, b2.reshape(1, HIDDEN // 2),
                W3, b3.reshape(1, 1))
